# baseline (device time: 6784 ns/iter reference)
import jax
import jax.numpy as jnp
from jax import lax
from jax.experimental import pallas as pl
from jax.experimental.pallas import tpu as pltpu

Y_SIZE = 2
BLK = 256


def kernel(x):
    m, n = x.shape

    HALF = BLK // 2

    def body(x_ref, out_ref, send_buf, send_sems, recv_sems):
        my_x = lax.axis_index("x")
        my_y = lax.axis_index("y")
        peer_y = 1 - my_y

        barrier_sem = pltpu.get_barrier_semaphore()
        pl.semaphore_signal(
            barrier_sem,
            inc=1,
            device_id=(my_x, peer_y),
            device_id_type=pl.DeviceIdType.MESH,
        )

        send_buf[pl.ds(0, HALF), :] = x_ref[
            pl.ds(0, HALF), pl.ds(peer_y * BLK, BLK)
        ].astype(jnp.bfloat16)

        pl.semaphore_wait(barrier_sem, 1)

        rdmas = []
        for i in range(2):
            rdmas.append(
                pltpu.make_async_remote_copy(
                    src_ref=send_buf.at[pl.ds(i * HALF, HALF), :],
                    dst_ref=out_ref.at[pl.ds(my_y * BLK + i * HALF, HALF), :],
                    send_sem=send_sems.at[i],
                    recv_sem=recv_sems.at[i],
                    device_id=(my_x, peer_y),
                    device_id_type=pl.DeviceIdType.MESH,
                )
            )
        rdmas[0].start()

        send_buf[pl.ds(HALF, HALF), :] = x_ref[
            pl.ds(HALF, HALF), pl.ds(peer_y * BLK, BLK)
        ].astype(jnp.bfloat16)
        rdmas[1].start()

        out_ref[pl.ds(my_y * BLK, BLK), :] = x_ref[
            :, pl.ds(my_y * BLK, BLK)
        ].astype(jnp.bfloat16)

        rdmas[0].wait()
        rdmas[1].wait()

    return pl.pallas_call(
        body,
        out_shape=jax.ShapeDtypeStruct((Y_SIZE * m, n // Y_SIZE), jnp.bfloat16),
        in_specs=[pl.BlockSpec(memory_space=pltpu.VMEM)],
        out_specs=pl.BlockSpec(memory_space=pltpu.VMEM),
        scratch_shapes=[
            pltpu.VMEM((m, BLK), jnp.bfloat16),
            pltpu.SemaphoreType.DMA((2,)),
            pltpu.SemaphoreType.DMA((2,)),
        ],
        compiler_params=pltpu.CompilerParams(collective_id=0),
    )(x)
